# baseline (speedup 1.0000x reference)
import functools

import jax
import jax.numpy as jnp
from jax import lax
from jax.experimental import pallas as pl
from jax.experimental.pallas import tpu as pltpu
from jax.experimental.pallas import tpu_sc as plsc

N_TOKENS = 8192
N_CODES = 8192
DIM = 256
COMMIT = 0.25

TN = 256
CH = 256
NCH = N_CODES // CH
FBIG = 3.0e38
NSPLIT = 2
NTOK_H = N_TOKENS // NSPLIT
NNT_H = NTOK_H // TN


def _argmin_body(x_ref, e_ref, x2_ref, e2_ref, idx_ref, sum_ref, acc_ref):
    n = pl.program_id(0)
    x2d = x_ref[...] * 2.0
    x2 = x2_ref[...]
    m_acc = None
    cc_acc = None
    for c in range(NCH):
        e_sub = e_ref[pl.ds(c * CH, CH), :]
        s2 = lax.dot_general(x2d, e_sub, (((1,), (1,)), ((), ())),
                             preferred_element_type=jnp.float32)
        d = (x2 + e2_ref[:, pl.ds(c * CH, CH)]) - s2
        d0 = d[:, :128]
        d1 = d[:, 128:]
        if m_acc is None:
            better = d1 < d0
            m_acc = jnp.where(better, d1, d0)
            cc_acc = jnp.where(better, 1.0, 0.0)
        else:
            b0 = d0 < m_acc
            cc_acc = jnp.where(b0, float(2 * c), cc_acc)
            m_acc = jnp.where(b0, d0, m_acc)
            b1 = d1 < m_acc
            cc_acc = jnp.where(b1, float(2 * c + 1), cc_acc)
            m_acc = jnp.where(b1, d1, m_acc)

    m_fin = jnp.min(m_acc, axis=1, keepdims=True)
    lane = lax.broadcasted_iota(jnp.int32, (TN, 128), 1).astype(jnp.float32)
    jfull = cc_acc * 128.0 + lane
    i_fin = jnp.min(jnp.where(m_acc == m_fin, jfull, FBIG),
                    axis=1, keepdims=True)
    idx_ref[...] = i_fin.astype(jnp.int32).reshape(1, 1, TN)
    prev = jnp.where(n == 0, 0.0, acc_ref[0])
    acc_ref[0] = prev + jnp.sum(m_fin)

    @pl.when(n == NNT_H - 1)
    def _():
        sum_ref[0] = acc_ref[0]


def _make_argmin_call(half):
    base = half * NNT_H
    return pl.pallas_call(
        _argmin_body,
        grid=(NNT_H,),
        in_specs=[
            pl.BlockSpec((TN, DIM), lambda n: (base + n, 0)),
            pl.BlockSpec((N_CODES, DIM), lambda n: (0, 0)),
            pl.BlockSpec((TN, 1), lambda n: (base + n, 0)),
            pl.BlockSpec((1, N_CODES), lambda n: (0, 0)),
        ],
        out_specs=[
            pl.BlockSpec((1, 1, TN), lambda n: (n, 0, 0)),
            pl.BlockSpec(memory_space=pltpu.SMEM),
        ],
        out_shape=[
            jax.ShapeDtypeStruct((NNT_H, 1, TN), jnp.int32),
            jax.ShapeDtypeStruct((1,), jnp.float32),
        ],
        scratch_shapes=[
            pltpu.SMEM((1,), jnp.float32),
        ],
    )


def _sc_gather(table, idx, ntok):
    mesh = plsc.VectorSubcoreMesh(core_axis_name="c", subcore_axis_name="s")
    nw = mesh.num_cores * mesh.num_subcores
    bpw = ntok // nw
    nch = bpw // 128
    idx3 = idx.reshape(nw, nch, 128)

    @functools.partial(
        pl.kernel,
        mesh=mesh,
        out_type=jax.ShapeDtypeStruct((ntok, DIM), jnp.float32),
        scratch_types=[
            pltpu.VMEM((nch, 128), jnp.int32),
            pltpu.VMEM((bpw, DIM), jnp.float32),
            pltpu.SemaphoreType.DMA,
        ],
    )
    def gather_kernel(table_hbm, idx_hbm, out_hbm, idx_v, rows_v, sem):
        wid = lax.axis_index("s") * mesh.num_cores + lax.axis_index("c")
        base = wid * bpw
        pltpu.sync_copy(idx_hbm.at[wid], idx_v)
        copies = [
            pltpu.async_copy(table_hbm.at[idx_v.at[j]],
                             rows_v.at[pl.ds(j * 128, 128)], sem)
            for j in range(nch)
        ]
        for c in copies:
            c.wait()
        pltpu.sync_copy(rows_v, out_hbm.at[pl.ds(base, bpw)])

    return gather_kernel(table, idx3)


def kernel(inputs, emb_weight):
    inputs = inputs.astype(jnp.float32)
    B, C, H, W = inputs.shape
    flat = jnp.transpose(inputs, (0, 2, 3, 1)).reshape(-1, DIM)
    x2 = jnp.sum(flat ** 2, axis=1, keepdims=True)
    e2 = jnp.sum(emb_weight.T ** 2, axis=0, keepdims=True)
    halves = []
    loss_sum = None
    for h in range(NSPLIT):
        idx3d, s_h = _make_argmin_call(h)(flat, emb_weight, x2, e2)
        q_h = _sc_gather(emb_weight, idx3d.reshape(-1), NTOK_H)
        halves.append(q_h)
        loss_sum = s_h[0] if loss_sum is None else loss_sum + s_h[0]
    q = jnp.concatenate(halves, axis=0)
    quantized_st = jnp.transpose(q.reshape(B, H, W, C), (0, 3, 1, 2))
    loss = loss_sum * (COMMIT / (N_TOKENS * DIM))
    return loss, quantized_st


# speedup vs baseline: 1.0634x; 1.0634x over previous
import functools

import jax
import jax.numpy as jnp
from jax import lax
from jax.experimental import pallas as pl
from jax.experimental.pallas import tpu as pltpu
from jax.experimental.pallas import tpu_sc as plsc

N_TOKENS = 8192
N_CODES = 8192
DIM = 256
COMMIT = 0.25

TN = 256
CH = 512
NNT = N_TOKENS // TN
NCH = N_CODES // CH
FBIG = 3.0e38


def _argmin_body(x_ref, e_ref, x2_ref, e2_ref, idx_ref, loss_ref, acc_ref):
    n = pl.program_id(0)
    x2d = x_ref[...] * 2.0
    x2 = x2_ref[...]
    m_acc = None
    cc_acc = None
    for c in range(NCH):
        e_sub = e_ref[pl.ds(c * CH, CH), :]
        s2 = lax.dot_general(x2d, e_sub, (((1,), (1,)), ((), ())),
                             preferred_element_type=jnp.float32)
        d = (x2 + e2_ref[:, pl.ds(c * CH, CH)]) - s2
        cols = [d[:, i * 128:(i + 1) * 128] for i in range(CH // 128)]
        for i, dc in enumerate(cols):
            cid = float(c * (CH // 128) + i)
            if m_acc is None:
                m_acc = dc
                cc_acc = jnp.zeros((TN, 128), jnp.float32)
            else:
                b = dc < m_acc
                cc_acc = jnp.where(b, cid, cc_acc)
                m_acc = jnp.where(b, dc, m_acc)

    m_fin = jnp.min(m_acc, axis=1, keepdims=True)
    lane = lax.broadcasted_iota(jnp.int32, (TN, 128), 1).astype(jnp.float32)
    jfull = cc_acc * 128.0 + lane
    i_fin = jnp.min(jnp.where(m_acc == m_fin, jfull, FBIG),
                    axis=1, keepdims=True)
    idx_ref[...] = i_fin.astype(jnp.int32).reshape(1, 1, TN)
    prev = jnp.where(n == 0, 0.0, acc_ref[0])
    acc_ref[0] = prev + jnp.sum(m_fin)

    @pl.when(n == NNT - 1)
    def _():
        loss_ref[0] = acc_ref[0] * (COMMIT / (N_TOKENS * DIM))


_argmin_call = pl.pallas_call(
    _argmin_body,
    grid=(NNT,),
    in_specs=[
        pl.BlockSpec((TN, DIM), lambda n: (n, 0)),
        pl.BlockSpec((N_CODES, DIM), lambda n: (0, 0)),
        pl.BlockSpec((TN, 1), lambda n: (n, 0)),
        pl.BlockSpec((1, N_CODES), lambda n: (0, 0)),
    ],
    out_specs=[
        pl.BlockSpec((1, 1, TN), lambda n: (n, 0, 0)),
        pl.BlockSpec(memory_space=pltpu.SMEM),
    ],
    out_shape=[
        jax.ShapeDtypeStruct((NNT, 1, TN), jnp.int32),
        jax.ShapeDtypeStruct((1,), jnp.float32),
    ],
    scratch_shapes=[
        pltpu.SMEM((1,), jnp.float32),
    ],
)


def _sc_gather(table, idx):
    mesh = plsc.VectorSubcoreMesh(core_axis_name="c", subcore_axis_name="s")
    nw = mesh.num_cores * mesh.num_subcores
    bpw = N_TOKENS // nw
    nch = bpw // 128
    idx3 = idx.reshape(nw, nch, 128)

    @functools.partial(
        pl.kernel,
        mesh=mesh,
        out_type=jax.ShapeDtypeStruct((N_TOKENS, DIM), jnp.float32),
        scratch_types=[
            pltpu.VMEM((nch, 128), jnp.int32),
            pltpu.VMEM((bpw, DIM), jnp.float32),
            pltpu.SemaphoreType.DMA,
        ],
    )
    def gather_kernel(table_hbm, idx_hbm, out_hbm, idx_v, rows_v, sem):
        wid = lax.axis_index("s") * mesh.num_cores + lax.axis_index("c")
        base = wid * bpw
        pltpu.sync_copy(idx_hbm.at[wid], idx_v)
        copies = [
            pltpu.async_copy(table_hbm.at[idx_v.at[j]],
                             rows_v.at[pl.ds(j * 128, 128)], sem)
            for j in range(nch)
        ]
        for c in copies:
            c.wait()
        pltpu.sync_copy(rows_v, out_hbm.at[pl.ds(base, bpw)])

    return gather_kernel(table, idx3)


def kernel(inputs, emb_weight):
    inputs = inputs.astype(jnp.float32)
    B, C, H, W = inputs.shape
    flat = jnp.transpose(inputs, (0, 2, 3, 1)).reshape(-1, DIM)
    x2 = jnp.sum(flat ** 2, axis=1, keepdims=True)
    e2 = jnp.sum(emb_weight.T ** 2, axis=0, keepdims=True)
    idx2d, loss1 = _argmin_call(flat, emb_weight, x2, e2)
    q = _sc_gather(emb_weight, idx2d.reshape(-1))
    quantized_st = jnp.transpose(q.reshape(B, H, W, C), (0, 3, 1, 2))
    return loss1[0], quantized_st


# speedup vs baseline: 1.0675x; 1.0039x over previous
import functools

import jax
import jax.numpy as jnp
from jax import lax
from jax.experimental import pallas as pl
from jax.experimental.pallas import tpu as pltpu
from jax.experimental.pallas import tpu_sc as plsc

N_TOKENS = 8192
N_CODES = 8192
DIM = 256
COMMIT = 0.25

TN = 256
CH = 256
NNT = N_TOKENS // TN
NCH = N_CODES // CH
FBIG = 3.0e38


def _argmin_body(x_ref, e_ref, x2_ref, e2_ref, idx_ref, loss_ref, acc_ref):
    n = pl.program_id(0)
    x2d = x_ref[...] * 2.0
    x2 = x2_ref[...]
    m_acc = None
    cc_acc = None
    for c in range(NCH):
        e_sub = e_ref[pl.ds(c * CH, CH), :]
        s2 = lax.dot_general(x2d, e_sub, (((1,), (1,)), ((), ())),
                             preferred_element_type=jnp.float32)
        d = (x2 + e2_ref[:, pl.ds(c * CH, CH)]) - s2
        d0 = d[:, :128]
        d1 = d[:, 128:]
        if m_acc is None:
            better = d1 < d0
            m_acc = jnp.where(better, d1, d0)
            cc_acc = jnp.where(better, 1.0, 0.0)
        else:
            b0 = d0 < m_acc
            cc_acc = jnp.where(b0, float(2 * c), cc_acc)
            m_acc = jnp.where(b0, d0, m_acc)
            b1 = d1 < m_acc
            cc_acc = jnp.where(b1, float(2 * c + 1), cc_acc)
            m_acc = jnp.where(b1, d1, m_acc)

    m_fin = jnp.min(m_acc, axis=1, keepdims=True)
    lane = lax.broadcasted_iota(jnp.int32, (TN, 128), 1).astype(jnp.float32)
    jfull = cc_acc * 128.0 + lane
    i_fin = jnp.min(jnp.where(m_acc == m_fin, jfull, FBIG),
                    axis=1, keepdims=True)
    idx_ref[...] = i_fin.astype(jnp.int32).reshape(1, 1, TN)
    prev = jnp.where(n == 0, 0.0, acc_ref[0])
    acc_ref[0] = prev + jnp.sum(m_fin)

    @pl.when(n == NNT - 1)
    def _():
        loss_ref[0] = acc_ref[0] * (COMMIT / (N_TOKENS * DIM))


_argmin_call = pl.pallas_call(
    _argmin_body,
    grid=(NNT,),
    in_specs=[
        pl.BlockSpec((TN, DIM), lambda n: (n, 0)),
        pl.BlockSpec((N_CODES, DIM), lambda n: (0, 0)),
        pl.BlockSpec((TN, 1), lambda n: (n, 0)),
        pl.BlockSpec((1, N_CODES), lambda n: (0, 0)),
    ],
    out_specs=[
        pl.BlockSpec((1, 1, TN), lambda n: (n, 0, 0)),
        pl.BlockSpec(memory_space=pltpu.SMEM),
    ],
    out_shape=[
        jax.ShapeDtypeStruct((NNT, 1, TN), jnp.int32),
        jax.ShapeDtypeStruct((1,), jnp.float32),
    ],
    scratch_shapes=[
        pltpu.SMEM((1,), jnp.float32),
    ],
)


def _sc_gather(table, idx):
    mesh = plsc.VectorSubcoreMesh(core_axis_name="c", subcore_axis_name="s")
    nw = mesh.num_cores * mesh.num_subcores
    bpw = N_TOKENS // nw
    nch = bpw // 128
    idx3 = idx.reshape(nw, nch, 128)

    @functools.partial(
        pl.kernel,
        mesh=mesh,
        out_type=jax.ShapeDtypeStruct((N_TOKENS, DIM), jnp.float32),
        scratch_types=[
            pltpu.VMEM((nch, 128), jnp.int32),
            pltpu.VMEM((bpw, DIM), jnp.float32),
            pltpu.SemaphoreType.DMA,
        ],
    )
    def gather_kernel(table_hbm, idx_hbm, out_hbm, idx_v, rows_v, sem):
        wid = lax.axis_index("s") * mesh.num_cores + lax.axis_index("c")
        base = wid * bpw
        pltpu.sync_copy(idx_hbm.at[wid], idx_v)
        copies = [
            pltpu.async_copy(table_hbm.at[idx_v.at[j]],
                             rows_v.at[pl.ds(j * 128, 128)], sem)
            for j in range(nch)
        ]
        for c in copies:
            c.wait()
        pltpu.sync_copy(rows_v, out_hbm.at[pl.ds(base, bpw)])

    return gather_kernel(table, idx3)


def kernel(inputs, emb_weight):
    inputs = inputs.astype(jnp.float32)
    B, C, H, W = inputs.shape
    flat = jnp.transpose(inputs, (0, 2, 3, 1)).reshape(-1, DIM)
    x2 = jnp.sum(flat ** 2, axis=1, keepdims=True)
    e2 = jnp.sum(emb_weight.T ** 2, axis=0, keepdims=True)
    idx2d, loss1 = _argmin_call(flat, emb_weight, x2, e2)
    q = _sc_gather(emb_weight, idx2d.reshape(-1))
    quantized_st = jnp.transpose(q.reshape(B, H, W, C), (0, 3, 1, 2))
    return loss1[0], quantized_st
